# baseline (speedup 1.0000x reference)
import functools

import jax
import jax.numpy as jnp
from jax import lax
from jax.experimental import pallas as pl
from jax.experimental.pallas import tpu as pltpu
from jax.experimental.pallas import tpu_sc as plsc

_L = 16
_NW = 32


def _sc_flip(B, K, C0, C1, H, W):
    P = H * W
    N0 = B * K * C0
    N1 = B * K * C1

    mesh = plsc.VectorSubcoreMesh(core_axis_name="c", subcore_axis_name="s")

    @functools.partial(
        pl.kernel,
        mesh=mesh,
        compiler_params=pltpu.CompilerParams(use_tc_tiling_on_sc=False),
        out_type=[
            jax.ShapeDtypeStruct((N0, P), jnp.float32),
            jax.ShapeDtypeStruct((N1, P), jnp.float32),
        ],
        scratch_types=[
            pltpu.VMEM((_NW,), jnp.int32),
            pltpu.VMEM((P,), jnp.float32),
            pltpu.VMEM((P,), jnp.float32),
        ],
    )
    def body(f0_hbm, f1_hbm, fi_hbm, o0_hbm, o1_hbm, fi_v, inb, outb):
        w = lax.axis_index("s") * 2 + lax.axis_index("c")
        pltpu.sync_copy(fi_hbm, fi_v)

        woffs = tuple(range(0, W - _L, _L)) + (W - _L,)

        def flip_plane(negate):
            def row(h, carry):
                rbase = h * W
                for woff in woffs:
                    v = inb[pl.ds(rbase + (W - _L - woff), _L)]
                    v = lax.rev(v, (0,))
                    outb[pl.ds(rbase + woff, _L)] = -v if negate else v
                return carry
            lax.fori_loop(0, H, row, 0)

        def do_field0(i, carry):
            p = w + _NW * i
            @pl.when(p < N0)
            def _():
                b = p // K
                k = p - b * K
                src = b * K + fi_v[pl.ds(k, _L)][0]
                pltpu.sync_copy(f0_hbm.at[src], inb)
                flip_plane(False)
                pltpu.sync_copy(outb, o0_hbm.at[p])
            return carry

        lax.fori_loop(0, -(-N0 // _NW), do_field0, 0)

        def do_field1(i, carry):
            p = w + _NW * i
            @pl.when(p < N1)
            def _():
                c = p % C1
                bk = p // C1
                b = bk // K
                k = bk - b * K
                src = (b * K + fi_v[pl.ds(k, _L)][0]) * C1 + c
                pltpu.sync_copy(f1_hbm.at[src], inb)

                @pl.when(c == 0)
                def _():
                    flip_plane(True)

                @pl.when(c != 0)
                def _():
                    flip_plane(False)

                pltpu.sync_copy(outb, o1_hbm.at[p])
            return carry

        lax.fori_loop(0, -(-N1 // _NW), do_field1, 0)

    return body


def kernel(field0, field1, flip_indices):
    B, K, C0, H, W = field0.shape
    C1 = field1.shape[2]
    P = H * W
    f0v = field0.reshape(B * K * C0, P)
    f1v = field1.reshape(B * K * C1, P)
    fi = jnp.pad(flip_indices.astype(jnp.int32), (0, _NW - flip_indices.shape[0]))

    o0, o1 = _sc_flip(B, K, C0, C1, H, W)(f0v, f1v, fi)
    return (o0.reshape(field0.shape), o1.reshape(field1.shape))


# speedup vs baseline: 3.8927x; 3.8927x over previous
import functools

import jax
import jax.numpy as jnp
from jax import lax
from jax.experimental import pallas as pl
from jax.experimental.pallas import tpu as pltpu
from jax.experimental.pallas import tpu_sc as plsc

_L = 16
_NW = 32


def _sc_flip(B, K, C0, C1, H, W):
    N0 = B * K * C0
    N1 = B * K * C1

    mesh = plsc.VectorSubcoreMesh(core_axis_name="c", subcore_axis_name="s")

    @functools.partial(
        pl.kernel,
        mesh=mesh,
        out_type=[
            jax.ShapeDtypeStruct((N0, H, W), jnp.float32),
            jax.ShapeDtypeStruct((N1, H, W), jnp.float32),
        ],
        scratch_types=[
            pltpu.VMEM((_NW,), jnp.int32),
            pltpu.VMEM((H, W), jnp.float32),
            pltpu.VMEM((H, W), jnp.float32),
        ],
    )
    def body(f0_hbm, f1_hbm, fi_hbm, o0_hbm, o1_hbm, fi_v, inb, outb):
        w = lax.axis_index("s") * 2 + lax.axis_index("c")
        pltpu.sync_copy(fi_hbm, fi_v)

        woffs = tuple(range(0, W - _L, _L)) + (W - _L,)

        def flip_plane(negate):
            def row(h, carry):
                for woff in woffs:
                    v = inb[h, pl.ds(W - _L - woff, _L)]
                    v = lax.rev(v, (0,))
                    outb[h, pl.ds(woff, _L)] = -v if negate else v
                return carry
            lax.fori_loop(0, H, row, 0)

        def do_field0(i, carry):
            p = w + _NW * i
            @pl.when(p < N0)
            def _():
                b = p // K
                k = p - b * K
                src = b * K + fi_v[pl.ds(k, _L)][0]
                pltpu.sync_copy(f0_hbm.at[src], inb)
                flip_plane(False)
                pltpu.sync_copy(outb, o0_hbm.at[p])
            return carry

        lax.fori_loop(0, -(-N0 // _NW), do_field0, 0)

        def do_field1(i, carry):
            p = w + _NW * i
            @pl.when(p < N1)
            def _():
                c = p % C1
                bk = p // C1
                b = bk // K
                k = bk - b * K
                src = (b * K + fi_v[pl.ds(k, _L)][0]) * C1 + c
                pltpu.sync_copy(f1_hbm.at[src], inb)

                @pl.when(c == 0)
                def _():
                    flip_plane(True)

                @pl.when(c != 0)
                def _():
                    flip_plane(False)

                pltpu.sync_copy(outb, o1_hbm.at[p])
            return carry

        lax.fori_loop(0, -(-N1 // _NW), do_field1, 0)

    return body


def kernel(field0, field1, flip_indices):
    B, K, C0, H, W = field0.shape
    C1 = field1.shape[2]
    f0v = field0.reshape(B * K * C0, H, W)
    f1v = field1.reshape(B * K * C1, H, W)
    fi = jnp.pad(flip_indices.astype(jnp.int32), (0, _NW - flip_indices.shape[0]))

    o0, o1 = _sc_flip(B, K, C0, C1, H, W)(f0v, f1v, fi)
    return (o0.reshape(field0.shape), o1.reshape(field1.shape))


# speedup vs baseline: 6.1294x; 1.5746x over previous
import functools

import jax
import jax.numpy as jnp
from jax import lax
from jax.experimental import layout as jex_layout
from jax.experimental import pallas as pl
from jax.experimental.pallas import tpu as pltpu
from jax.experimental.pallas import tpu_sc as plsc

_L = 16
_NW = 32


def _sc_flip(B, K, C0, C1, H, W):
    N0 = B * K * C0
    N1 = B * K * C1

    mesh = plsc.VectorSubcoreMesh(core_axis_name="c", subcore_axis_name="s")

    @functools.partial(
        pl.kernel,
        mesh=mesh,
        out_type=[
            jax.ShapeDtypeStruct((N0, H, W), jnp.float32),
            jax.ShapeDtypeStruct((N1, H, W), jnp.float32),
        ],
        scratch_types=[
            pltpu.VMEM((_NW,), jnp.int32),
            pltpu.VMEM((H, W), jnp.float32),
            pltpu.VMEM((H, W), jnp.float32),
        ],
    )
    def body(f0_hbm, f1_hbm, fi_hbm, o0_hbm, o1_hbm, fi_v, inb, outb):
        w = lax.axis_index("s") * 2 + lax.axis_index("c")
        pltpu.sync_copy(fi_hbm, fi_v)

        woffs = tuple(range(0, W - _L, _L)) + (W - _L,)

        def flip_plane(negate):
            def row(h, carry):
                for woff in woffs:
                    v = inb[h, pl.ds(W - _L - woff, _L)]
                    v = lax.rev(v, (0,))
                    outb[h, pl.ds(woff, _L)] = -v if negate else v
                return carry
            lax.fori_loop(0, H, row, 0)

        def do_field0(i, carry):
            p = w + _NW * i
            @pl.when(p < N0)
            def _():
                b = p // K
                k = p - b * K
                src = b * K + fi_v[pl.ds(k, _L)][0]
                pltpu.sync_copy(f0_hbm.at[src], inb)
                flip_plane(False)
                pltpu.sync_copy(outb, o0_hbm.at[p])
            return carry

        lax.fori_loop(0, -(-N0 // _NW), do_field0, 0)

        def do_field1(i, carry):
            p = w + _NW * i
            @pl.when(p < N1)
            def _():
                c = p % C1
                bk = p // C1
                b = bk // K
                k = bk - b * K
                src = (b * K + fi_v[pl.ds(k, _L)][0]) * C1 + c
                pltpu.sync_copy(f1_hbm.at[src], inb)

                @pl.when(c == 0)
                def _():
                    flip_plane(True)

                @pl.when(c != 0)
                def _():
                    flip_plane(False)

                pltpu.sync_copy(outb, o1_hbm.at[p])
            return carry

        lax.fori_loop(0, -(-N1 // _NW), do_field1, 0)

    return body


def kernel(field0, field1, flip_indices):
    B, K, C0, H, W = field0.shape
    C1 = field1.shape[2]
    lay5 = jex_layout.Layout(major_to_minor=(0, 1, 2, 3, 4))
    field0, field1 = lax.optimization_barrier((field0, field1))
    field0 = jex_layout.with_layout_constraint(field0, lay5)
    field1 = jex_layout.with_layout_constraint(field1, lay5)
    f0v = field0.reshape(B * K * C0, H, W)
    f1v = field1.reshape(B * K * C1, H, W)
    fi = jnp.pad(flip_indices.astype(jnp.int32), (0, _NW - flip_indices.shape[0]))

    o0, o1 = _sc_flip(B, K, C0, C1, H, W)(f0v, f1v, fi)
    o0 = jex_layout.with_layout_constraint(o0.reshape(field0.shape), lay5)
    o1 = jex_layout.with_layout_constraint(o1.reshape(field1.shape), lay5)
    return (o0, o1)
